# baseline (speedup 1.0000x reference)
import functools
import numpy as np
import jax
import jax.numpy as jnp
from jax import lax
from jax.experimental import pallas as pl
from jax.experimental.pallas import tpu as pltpu
from jax.experimental.pallas import tpu_sc as plsc

_IN_D = 74
_OUT_D = 157
_B = 1024
_S = 256
_NW = 32
_B_PER_W = _B // _NW
_S_CHUNK = 128
_CHUNKS_PER_B = _S // _S_CHUNK
_N_CHUNK = _B_PER_W * _CHUNKS_PER_B
_GROUPS = _S_CHUNK // 16

_BUCKETS = [
    (14, 20, True, 1500.0),
    (15, 20, True, 1500.0),
    (19, 16, True, 3000.0),
    (56, 16, False, 120.0),
    (57, 16, False, 120.0),
]
_BUCKET_COLS = frozenset(c for c, _, _, _ in _BUCKETS)
_PASSTHROUGH = [c for c in range(_IN_D) if c not in _BUCKET_COLS]


def _bucket_plan():
    pass_pairs = [(j, s) for j, s in enumerate(_PASSTHROUGH)]
    onehot = []
    out_c = len(_PASSTHROUGH)
    for src, nb, is_sqrt, mx in _BUCKETS:
        if is_sqrt:
            thr = [(t / (nb - 1)) ** 2 * mx for t in range(nb)]
        else:
            thr = [t / (nb - 1) * mx for t in range(nb)]
        lo = [-np.inf] + [np.float32(t) for t in thr[1:]]
        hi = [np.float32(t) for t in thr[1:]] + [np.inf]
        for t in range(nb):
            onehot.append((out_c, src, float(lo[t]), float(hi[t])))
            out_c += 1
    assert out_c == _OUT_D
    return pass_pairs, onehot


_PASS_PAIRS, _ONEHOT_COLS = _bucket_plan()


def _compute_chunk(in_ref, out_ref):
    iota = lax.iota(jnp.int32, 16)

    def col(c):
        return jnp.full((16,), c, dtype=jnp.int32)

    def group_body(g, carry):
        rows = iota + g * 16
        src_vecs = {}
        for src, _, _, _ in _BUCKETS:
            if src not in src_vecs:
                src_vecs[src] = plsc.load_gather(in_ref, [rows, col(src)])
        for out_c, src_c in _PASS_PAIRS:
            v = plsc.load_gather(in_ref, [rows, col(src_c)])
            plsc.store_scatter(out_ref, [rows, col(out_c)], v)
        one = jnp.float32(1.0)
        zero = jnp.float32(0.0)
        for out_c, src_c, lo, hi in _ONEHOT_COLS:
            v = src_vecs[src_c]
            if np.isinf(lo):
                m = v < hi
            elif np.isinf(hi):
                m = v >= lo
            else:
                m = (v >= lo) & (v < hi)
            plsc.store_scatter(out_ref, [rows, col(out_c)], jnp.where(m, one, zero))
        return carry

    lax.fori_loop(0, _GROUPS, group_body, 0)


def _sc_body(in_hbm, out_hbm, in_buf0, in_buf1, out_buf0, out_buf1,
             in_sem0, in_sem1, out_sem0, out_sem1):
    nc = 2
    wid = lax.axis_index("s") * nc + lax.axis_index("c")
    bfirst = wid * _B_PER_W
    in_bufs = [in_buf0, in_buf1]
    out_bufs = [out_buf0, out_buf1]
    in_sems = [in_sem0, in_sem1]
    out_sems = [out_sem0, out_sem1]

    def in_slice(ci):
        b = bfirst + ci // _CHUNKS_PER_B
        s0 = (ci % _CHUNKS_PER_B) * _S_CHUNK
        return in_hbm.at[b, pl.ds(s0, _S_CHUNK), :]

    def out_slice(ci):
        b = bfirst + ci // _CHUNKS_PER_B
        s0 = (ci % _CHUNKS_PER_B) * _S_CHUNK
        return out_hbm.at[b, pl.ds(s0, _S_CHUNK), :]

    pltpu.async_copy(in_slice(0), in_bufs[0], in_sems[0])
    pltpu.async_copy(in_slice(1), in_bufs[1], in_sems[1])

    def outer(i, carry):
        for b in range(2):
            ci = i * 2 + b
            pltpu.make_async_copy(in_slice(ci), in_bufs[b], in_sems[b]).wait()
            @pl.when(i >= 1)
            def _():
                pltpu.make_async_copy(out_bufs[b], out_slice(ci),
                                      out_sems[b]).wait()
            _compute_chunk(in_bufs[b], out_bufs[b])
            @pl.when(i < _N_CHUNK // 2 - 1)
            def _():
                pltpu.async_copy(in_slice(ci + 2), in_bufs[b], in_sems[b])
            pltpu.async_copy(out_bufs[b], out_slice(ci), out_sems[b])
        return carry

    lax.fori_loop(0, _N_CHUNK // 2, outer, 0)
    for b in range(2):
        ci = _N_CHUNK - 2 + b
        pltpu.make_async_copy(out_bufs[b], out_slice(ci), out_sems[b]).wait()


@jax.jit
def _preprocess(features):
    mesh = plsc.VectorSubcoreMesh(core_axis_name="c", subcore_axis_name="s")
    k = pl.kernel(
        _sc_body,
        out_type=jax.ShapeDtypeStruct((_B, _S, _OUT_D), jnp.float32),
        mesh=mesh,
        scratch_types=[
            pltpu.VMEM((_S_CHUNK, _IN_D), jnp.float32),
            pltpu.VMEM((_S_CHUNK, _IN_D), jnp.float32),
            pltpu.VMEM((_S_CHUNK, _OUT_D), jnp.float32),
            pltpu.VMEM((_S_CHUNK, _OUT_D), jnp.float32),
            pltpu.SemaphoreType.DMA,
            pltpu.SemaphoreType.DMA,
            pltpu.SemaphoreType.DMA,
            pltpu.SemaphoreType.DMA,
        ],
        compiler_params=pltpu.CompilerParams(needs_layout_passes=False),
    )
    return k(features)


def kernel(features):
    return _preprocess(features.astype(jnp.float32))


# speedup vs baseline: 2.1887x; 2.1887x over previous
import functools
import numpy as np
import jax
import jax.numpy as jnp
from jax import lax
from jax.experimental import pallas as pl
from jax.experimental.pallas import tpu as pltpu
from jax.experimental.pallas import tpu_sc as plsc

_IN_D = 74
_OUT_D = 157
_B = 1024
_S = 256
_NW = 32
_B_PER_W = _B // _NW
_S_CHUNK = 128
_CHUNKS_PER_B = _S // _S_CHUNK
_N_CHUNK = _B_PER_W * _CHUNKS_PER_B

_BUCKETS = [
    (14, 20, True, 1500.0),
    (15, 20, True, 1500.0),
    (19, 16, True, 3000.0),
    (56, 16, False, 120.0),
    (57, 16, False, 120.0),
]
_BUCKET_COLS = frozenset(c for c, _, _, _ in _BUCKETS)
_PASSTHROUGH = [c for c in range(_IN_D) if c not in _BUCKET_COLS]
_N_PASS = len(_PASSTHROUGH)

_PASS_RUNS = []
_run_dst = 0
_run_src = _PASSTHROUGH[0]
_prev = _PASSTHROUGH[0] - 1
for _j, _s in enumerate(_PASSTHROUGH):
    if _s != _prev + 1:
        _PASS_RUNS.append((_run_dst, _run_src, _j - _run_dst))
        _run_dst, _run_src = _j, _s
    _prev = _s
_PASS_RUNS.append((_run_dst, _run_src, _N_PASS - _run_dst))

_OH_BOUNDS = []
for _src, _nb, _is_sqrt, _mx in _BUCKETS:
    if _is_sqrt:
        _thr = [(t / (_nb - 1)) ** 2 * _mx for t in range(_nb)]
    else:
        _thr = [t / (_nb - 1) * _mx for t in range(_nb)]
    _lo = [-np.inf] + [np.float32(t) for t in _thr[1:]]
    _hi = [np.float32(t) for t in _thr[1:]] + [np.inf]
    for _t in range(_nb):
        _OH_BOUNDS.append((float(_lo[_t]), float(_hi[_t]), _src))
assert _N_PASS + len(_OH_BOUNDS) == _OUT_D

_OH_WINDOWS = []
for _seg_lo, _seg_hi in ((_N_PASS, 128), (128, _OUT_D)):
    _c0 = _seg_lo
    while _c0 + 16 <= _seg_hi:
        _OH_WINDOWS.append(_c0)
        _c0 += 16
    if _c0 < _seg_hi:
        _OH_WINDOWS.append(_seg_hi - 16)


_BLOCK_BY_SRC = {}
_bc = _N_PASS
for _src, _nb, _is_sqrt, _mx in _BUCKETS:
    _BLOCK_BY_SRC[_src] = (_bc, _nb, _is_sqrt, _mx)
    _bc += _nb


def _oh_window_plan(c0):
    srcs = []
    lanes_src = []
    inf_lo_lanes = []
    inf_hi_lanes = []
    for l in range(16):
        b_lo, b_hi, b_src = _OH_BOUNDS[c0 + l - _N_PASS]
        if b_src not in srcs:
            srcs.append(b_src)
        lanes_src.append(b_src)
        if np.isinf(b_lo):
            inf_lo_lanes.append(l)
        if np.isinf(b_hi):
            inf_hi_lanes.append(l)
    first_src = srcs[0]
    n_first = sum(1 for s in lanes_src if s == first_src)
    assert lanes_src == [first_src] * n_first + [srcs[-1]] * (16 - n_first)
    return srcs, n_first, inf_lo_lanes, inf_hi_lanes


_OH_PLANS = [(_c, _oh_window_plan(_c)) for _c in _OH_WINDOWS]


def _compute_chunk(in_ref, out_ref):
    iota = lax.iota(jnp.int32, 16)
    iota_f = iota.astype(jnp.float32)
    one = jnp.float32(1.0)
    zero = jnp.float32(0.0)
    neg_inf = jnp.float32(-np.inf)
    pos_inf = jnp.float32(np.inf)

    def block_bounds(src, c0):
        bstart, nb, is_sqrt, mx = _BLOCK_BY_SRC[src]
        t = iota_f + jnp.float32(c0 - bstart)
        if is_sqrt:
            s = np.sqrt(mx) / (nb - 1)
            lo = (t * jnp.float32(s)) * (t * jnp.float32(s))
            t1 = t + one
            hi = (t1 * jnp.float32(s)) * (t1 * jnp.float32(s))
        else:
            s = mx / (nb - 1)
            lo = t * jnp.float32(s)
            hi = lo + jnp.float32(s)
        return lo, hi

    window_consts = []
    for c0, (srcs, n_first, inf_lo, inf_hi) in _OH_PLANS:
        lo, hi = block_bounds(srcs[0], c0)
        if len(srcs) > 1:
            lo_b, hi_b = block_bounds(srcs[1], c0)
            first_mask = iota < n_first
            lo = jnp.where(first_mask, lo, lo_b)
            hi = jnp.where(first_mask, hi, hi_b)
        for l in inf_lo:
            lo = jnp.where(iota == l, neg_inf, lo)
        for l in inf_hi:
            hi = jnp.where(iota == l, pos_inf, hi)
        window_consts.append((c0, srcs, n_first, lo, hi))

    def row_body(r, carry):
        for dst0, src0, length in _PASS_RUNS:
            for k in range(0, length, 16):
                s0 = min(src0 + k, _IN_D - 16)
                d0 = s0 - src0 + dst0
                out_ref[r, pl.ds(d0, 16)] = in_ref[r, pl.ds(s0, 16)]
        vec_a = in_ref[r, pl.ds(14, 16)]
        vec_b = in_ref[r, pl.ds(56, 16)]
        splats = {
            14: jnp.full((16,), vec_a[0], dtype=jnp.float32),
            15: jnp.full((16,), vec_a[1], dtype=jnp.float32),
            19: jnp.full((16,), vec_a[5], dtype=jnp.float32),
            56: jnp.full((16,), vec_b[0], dtype=jnp.float32),
            57: jnp.full((16,), vec_b[1], dtype=jnp.float32),
        }
        for c0, srcs, n_first, lo, hi in window_consts:
            v = splats[srcs[0]]
            if len(srcs) > 1:
                v = jnp.where(iota < n_first, v, splats[srcs[1]])
            m = (v >= lo) & (v < hi)
            out_ref[r, pl.ds(c0, 16)] = jnp.where(m, one, zero)
        return carry

    lax.fori_loop(0, _S_CHUNK, row_body, 0)


def _sc_body(in_hbm, out_hbm, in_buf0, in_buf1, out_buf0, out_buf1,
             in_sem0, in_sem1, out_sem0, out_sem1):
    nc = 2
    wid = lax.axis_index("s") * nc + lax.axis_index("c")
    bfirst = wid * _B_PER_W
    in_bufs = [in_buf0, in_buf1]
    out_bufs = [out_buf0, out_buf1]
    in_sems = [in_sem0, in_sem1]
    out_sems = [out_sem0, out_sem1]

    def in_slice(ci):
        b = bfirst + ci // _CHUNKS_PER_B
        s0 = (ci % _CHUNKS_PER_B) * _S_CHUNK
        return in_hbm.at[b, pl.ds(s0, _S_CHUNK), :]

    def out_slice(ci):
        b = bfirst + ci // _CHUNKS_PER_B
        s0 = (ci % _CHUNKS_PER_B) * _S_CHUNK
        return out_hbm.at[b, pl.ds(s0, _S_CHUNK), :]

    pltpu.async_copy(in_slice(0), in_bufs[0], in_sems[0])
    pltpu.async_copy(in_slice(1), in_bufs[1], in_sems[1])

    def outer(i, carry):
        for b in range(2):
            ci = i * 2 + b
            pltpu.make_async_copy(in_slice(ci), in_bufs[b], in_sems[b]).wait()
            @pl.when(i >= 1)
            def _():
                pltpu.make_async_copy(out_bufs[b], out_slice(ci),
                                      out_sems[b]).wait()
            _compute_chunk(in_bufs[b], out_bufs[b])
            @pl.when(i < _N_CHUNK // 2 - 1)
            def _():
                pltpu.async_copy(in_slice(ci + 2), in_bufs[b], in_sems[b])
            pltpu.async_copy(out_bufs[b], out_slice(ci), out_sems[b])
        return carry

    lax.fori_loop(0, _N_CHUNK // 2, outer, 0)
    for b in range(2):
        ci = _N_CHUNK - 2 + b
        pltpu.make_async_copy(out_bufs[b], out_slice(ci), out_sems[b]).wait()


@jax.jit
def _preprocess(features):
    mesh = plsc.VectorSubcoreMesh(core_axis_name="c", subcore_axis_name="s")
    k = pl.kernel(
        _sc_body,
        out_type=jax.ShapeDtypeStruct((_B, _S, _OUT_D), jnp.float32),
        mesh=mesh,
        scratch_types=[
            pltpu.VMEM((_S_CHUNK, _IN_D), jnp.float32),
            pltpu.VMEM((_S_CHUNK, _IN_D), jnp.float32),
            pltpu.VMEM((_S_CHUNK, _OUT_D), jnp.float32),
            pltpu.VMEM((_S_CHUNK, _OUT_D), jnp.float32),
            pltpu.SemaphoreType.DMA,
            pltpu.SemaphoreType.DMA,
            pltpu.SemaphoreType.DMA,
            pltpu.SemaphoreType.DMA,
        ],
        compiler_params=pltpu.CompilerParams(needs_layout_passes=False),
    )
    return k(features)


def kernel(features):
    return _preprocess(features)


# speedup vs baseline: 5.1586x; 2.3569x over previous
import functools
import numpy as np
import jax
import jax.numpy as jnp
from jax import lax
from jax.experimental import pallas as pl
from jax.experimental.pallas import tpu as pltpu
from jax.experimental.pallas import tpu_sc as plsc

_IN_D = 74
_OUT_D = 157
_B = 1024
_S = 256
_NW = 32
_R_PER_W = _B // _NW
_SLAB_W = _R_PER_W * _S
_VECS = _SLAB_W // 16

_BUCKETS = [
    (14, 20, True, 1500.0),
    (15, 20, True, 1500.0),
    (19, 16, True, 3000.0),
    (56, 16, False, 120.0),
    (57, 16, False, 120.0),
]
_BUCKET_COLS = [c for c, _, _, _ in _BUCKETS]
_PASSTHROUGH = [c for c in range(_IN_D) if c not in _BUCKET_COLS]
_N_PASS = len(_PASSTHROUGH)

_OH_PLAN = []
_out_p = _N_PASS
for _slot, (_src, _nb, _is_sqrt, _mx) in enumerate(_BUCKETS):
    if _is_sqrt:
        _thr = [(t / (_nb - 1)) ** 2 * _mx for t in range(_nb)]
    else:
        _thr = [t / (_nb - 1) * _mx for t in range(_nb)]
    _lo = [-np.inf] + [float(np.float32(t)) for t in _thr[1:]]
    _hi = [float(np.float32(t)) for t in _thr[1:]] + [np.inf]
    for _t in range(_nb):
        _OH_PLAN.append((_out_p, _slot, _lo[_t], _hi[_t]))
        _out_p += 1
assert _out_p == _OUT_D


def _sc_body(in_hbm, out_hbm, src0, src1, src2, src3, src4,
             pass_buf0, pass_buf1, pass_buf2, pass_buf3, out_buf0, out_buf1,
             src_sem, pass_sem0, pass_sem1, pass_sem2, pass_sem3,
             out_sem0, out_sem1):
    nc = 2
    wid = lax.axis_index("s") * nc + lax.axis_index("c")
    r0 = wid * _R_PER_W
    src_bufs = [src0, src1, src2, src3, src4]
    pass_bufs = [pass_buf0, pass_buf1, pass_buf2, pass_buf3]
    pass_sems = [pass_sem0, pass_sem1, pass_sem2, pass_sem3]
    out_bufs = [out_buf0, out_buf1]
    out_sems = [out_sem0, out_sem1]

    def in_slab(p):
        return in_hbm.at[p, pl.ds(r0, _R_PER_W), :]

    def out_slab(p):
        return out_hbm.at[p, pl.ds(r0, _R_PER_W), :]

    for i, (src_c, _, _, _) in enumerate(_BUCKETS):
        pltpu.async_copy(in_slab(src_c), src_bufs[i], src_sem)
    for i, (src_c, _, _, _) in enumerate(_BUCKETS):
        pltpu.make_async_copy(in_slab(src_c), src_bufs[i], src_sem).wait()

    def pass_wait_in(j):
        b = j % 4
        pltpu.make_async_copy(in_slab(_PASSTHROUGH[j]), pass_bufs[b],
                              pass_sems[b]).wait()

    def pass_wait_out(j):
        b = j % 4
        pltpu.make_async_copy(pass_bufs[b], out_slab(j), pass_sems[b]).wait()

    for j in range(_N_PASS):
        b = j % 4
        if j >= 4:
            pass_wait_out(j - 4)
        pltpu.async_copy(in_slab(_PASSTHROUGH[j]), pass_bufs[b], pass_sems[b])
        if j >= 2:
            b2 = (j - 2) % 4
            pass_wait_in(j - 2)
            pltpu.async_copy(pass_bufs[b2], out_slab(j - 2), pass_sems[b2])
    for j in (_N_PASS - 2, _N_PASS - 1):
        pass_wait_in(j)
        pltpu.async_copy(pass_bufs[j % 4], out_slab(j), pass_sems[j % 4])
    for j in range(_N_PASS - 4, _N_PASS):
        pass_wait_out(j)

    def compute_plane(slot, lo, hi, dst):
        src = src_bufs[slot]

        def vec_body(i, carry):
            r = i // 16
            c = (i % 16) * 16
            v = src[r, pl.ds(c, 16)]
            if np.isinf(lo):
                m = v < jnp.float32(hi)
            elif np.isinf(hi):
                m = v >= jnp.float32(lo)
            else:
                m = (v >= jnp.float32(lo)) & (v < jnp.float32(hi))
            dst[r, pl.ds(c, 16)] = jnp.where(m, jnp.float32(1.0),
                                             jnp.float32(0.0))
            return carry

        lax.fori_loop(0, _VECS, vec_body, 0)

    for k, (out_p, slot, lo, hi) in enumerate(_OH_PLAN):
        b = k % 2
        if k >= 2:
            prev_p = _OH_PLAN[k - 2][0]
            pltpu.make_async_copy(out_bufs[b], out_slab(prev_p),
                                  out_sems[b]).wait()
        compute_plane(slot, lo, hi, out_bufs[b])
        pltpu.async_copy(out_bufs[b], out_slab(out_p), out_sems[b])
    for k in (len(_OH_PLAN) - 2, len(_OH_PLAN) - 1):
        b = k % 2
        out_p = _OH_PLAN[k][0]
        pltpu.make_async_copy(out_bufs[b], out_slab(out_p), out_sems[b]).wait()


@jax.jit
def _preprocess(planes):
    mesh = plsc.VectorSubcoreMesh(core_axis_name="c", subcore_axis_name="s")
    k = pl.kernel(
        _sc_body,
        out_type=jax.ShapeDtypeStruct((_OUT_D, _B, _S), jnp.float32),
        mesh=mesh,
        scratch_types=[
            pltpu.VMEM((_R_PER_W, _S), jnp.float32),
            pltpu.VMEM((_R_PER_W, _S), jnp.float32),
            pltpu.VMEM((_R_PER_W, _S), jnp.float32),
            pltpu.VMEM((_R_PER_W, _S), jnp.float32),
            pltpu.VMEM((_R_PER_W, _S), jnp.float32),
            pltpu.VMEM((_R_PER_W, _S), jnp.float32),
            pltpu.VMEM((_R_PER_W, _S), jnp.float32),
            pltpu.VMEM((_R_PER_W, _S), jnp.float32),
            pltpu.VMEM((_R_PER_W, _S), jnp.float32),
            pltpu.VMEM((_R_PER_W, _S), jnp.float32),
            pltpu.VMEM((_R_PER_W, _S), jnp.float32),
            pltpu.SemaphoreType.DMA,
            pltpu.SemaphoreType.DMA,
            pltpu.SemaphoreType.DMA,
            pltpu.SemaphoreType.DMA,
            pltpu.SemaphoreType.DMA,
            pltpu.SemaphoreType.DMA,
            pltpu.SemaphoreType.DMA,
        ],
        compiler_params=pltpu.CompilerParams(needs_layout_passes=True),
    )
    return k(planes)


def kernel(features):
    planes = jnp.transpose(features, (2, 0, 1))
    out_planes = _preprocess(planes)
    return jnp.transpose(out_planes, (1, 2, 0))
